# baseline (speedup 1.0000x reference)
import functools

import jax
import jax.numpy as jnp
from jax import lax
from jax.experimental import pallas as pl
from jax.experimental.pallas import tpu as pltpu
from jax.experimental.pallas import tpu_sc as plsc

N = 10000
E = 320000
NB = 128
NF = 128
NG = 32
NC = 3
NMOL = 100
CUTOFF = 5.0

LOG2 = 0.6931471805599453

SC_CORES = 2
SC_TILES = 16
NW = SC_CORES * SC_TILES
EPAD = 327680
EPW = EPAD // NW
CHUNK = 64
NCHUNK = EPW // CHUNK
NROWS = 10048
ROWS_PT = NROWS // SC_TILES
ZROWS = 8

_sc_mesh = plsc.VectorSubcoreMesh(core_axis_name="c", subcore_axis_name="s")
_sc_params = pltpu.CompilerParams(needs_layout_passes=False, use_tc_tiling_on_sc=False)


def _ssp(x):
    return jnp.maximum(x, 0.0) + jnp.log1p(jnp.exp(-jnp.abs(x))) - LOG2


def _dist_body(xs_hbm, ys_hbm, zs_hbm, a0_hbm, a1_hbm, out_hbm, xs_v, ys_v, zs_v, i0_v, i1_v, e2_v):
    cid = lax.axis_index("c")
    sid = lax.axis_index("s")
    wid = cid * SC_TILES + sid

    pltpu.sync_copy(xs_hbm, xs_v)
    pltpu.sync_copy(ys_hbm, ys_v)
    pltpu.sync_copy(zs_hbm, zs_v)
    pltpu.sync_copy(a0_hbm.at[wid], i0_v)
    pltpu.sync_copy(a1_hbm.at[wid], i1_v)

    def step(i, carry):
        a0 = i0_v[i, :]
        a1 = i1_v[i, :]
        dx = plsc.load_gather(xs_v, [a0]) - plsc.load_gather(xs_v, [a1])
        dy = plsc.load_gather(ys_v, [a0]) - plsc.load_gather(ys_v, [a1])
        dz = plsc.load_gather(zs_v, [a0]) - plsc.load_gather(zs_v, [a1])
        e2_v[i, :] = dx * dx + dy * dy + dz * dz
        return carry

    lax.fori_loop(0, EPW // 16, step, 0)
    pltpu.sync_copy(e2_v, out_hbm.at[wid])


def _dist_call(xs, ys, zs, a0r, a1r):
    kfn = pl.kernel(
        _dist_body,
        out_type=jax.ShapeDtypeStruct((NW, EPW // 16, 16), jnp.float32),
        mesh=_sc_mesh,
        compiler_params=_sc_params,
        scratch_types=[
            pltpu.VMEM((NROWS,), jnp.float32),
            pltpu.VMEM((NROWS,), jnp.float32),
            pltpu.VMEM((NROWS,), jnp.float32),
            pltpu.VMEM((EPW // 16, 16), jnp.int32),
            pltpu.VMEM((EPW // 16, 16), jnp.int32),
            pltpu.VMEM((EPW // 16, 16), jnp.float32),
        ],
    )
    return kfn(xs, ys, zs, a0r, a1r)


def _conv_body(h_hbm, wi_hbm, a0_hbm, a1_hbm, out_hbm, agg_sh,
               i0a, i0b, i1a, i1b, hg0, hg1, hg2, wv0, wv1, wv2,
               sx0, sx1, sx2, zb_v,
               sg0, sg1, sg2, sw0, sw1, sw2, ss0, ss1, ss2, q0a, q0b, q1a, q1b):
    cid = lax.axis_index("c")
    sid = lax.axis_index("s")
    wid = cid * SC_TILES + sid

    def zrow(r, carry):
        for c in range(NF // 16):
            zb_v[r, pl.ds(c * 16, 16)] = jnp.zeros((16,), jnp.float32)
        return carry

    lax.fori_loop(0, ZROWS, zrow, 0)

    row0 = sid * ROWS_PT

    def zblk(k, carry):
        pltpu.sync_copy(zb_v, agg_sh.at[pl.ds(row0 + k * ZROWS, ZROWS)])
        return carry

    lax.fori_loop(0, ROWS_PT // ZROWS, zblk, 0)

    if ROWS_PT % ZROWS != 0:
        tail = ROWS_PT % ZROWS
        pltpu.sync_copy(zb_v.at[pl.ds(0, tail)],
                        agg_sh.at[pl.ds(row0 + (ROWS_PT // ZROWS) * ZROWS, tail)])

    plsc.subcore_barrier()

    ebase = wid * EPW
    ibufs = ((i0a, i1a, q0a, q1a), (i0b, i1b, q0b, q1b))
    dbufs = ((hg0, wv0, sx0, sg0, sw0, ss0),
             (hg1, wv1, sx1, sg1, sw1, ss1),
             (hg2, wv2, sx2, sg2, sw2, ss2))

    def issue_idx(ch, p2):
        i0, i1, q0, q1 = ibufs[p2]
        pltpu.async_copy(a0_hbm.at[wid, ch], i0, q0)
        pltpu.async_copy(a1_hbm.at[wid, ch], i1, q1)

    def wait_idx(ch, p2):
        i0, i1, q0, q1 = ibufs[p2]
        pltpu.make_async_copy(a0_hbm.at[wid, ch], i0, q0).wait()
        pltpu.make_async_copy(a1_hbm.at[wid, ch], i1, q1).wait()

    def issue_main(ch, p2, p3):
        i1 = ibufs[p2][1]
        hg, wv, _, sg, sw, _ = dbufs[p3]
        pltpu.async_copy(h_hbm.at[i1], hg, sg)
        pltpu.async_copy(wi_hbm.at[pl.ds(ebase + ch * CHUNK, CHUNK)], wv, sw)

    def wait_main(ch, p2, p3):
        i1 = ibufs[p2][1]
        hg, wv, _, sg, sw, _ = dbufs[p3]
        pltpu.make_async_copy(h_hbm.at[i1], hg, sg).wait()
        pltpu.make_async_copy(wi_hbm.at[pl.ds(ebase + ch * CHUNK, CHUNK)], wv, sw).wait()

    def wait_scatter(p3):
        hg, wv, sx, _, _, ss = dbufs[p3]
        pltpu.make_async_copy(hg, agg_sh.at[sx], ss).wait()

    def mult_scatter(p2, p3):
        i0 = ibufs[p2][0]
        hg, wv, sx, _, _, ss = dbufs[p3]

        def mrow(r, c2):
            for c in range(NF // 32):
                we, wo = plsc.unpack(wv[r, pl.ds(c * 32, 32)],
                                     format=plsc.PackFormat.INTERLEAVED)
                se = pl.ds(c * 32, 16)
                so = pl.ds(c * 32 + 16, 16)
                hg[r, se] = we * hg[r, se]
                hg[r, so] = wo * hg[r, so]
            return c2

        lax.fori_loop(0, CHUNK, mrow, 0)
        for c in range(CHUNK // 16):
            s = pl.ds(c * 16, 16)
            sx[s] = i0[s]
        pltpu.async_copy(hg, agg_sh.at[sx], ss, add=True)

    def step(k, i):
        p3 = i % 3
        p2 = i % 2

        if isinstance(k, int):
            if k >= 2:
                wait_scatter((k - 2) % 3)
            if k + 1 < NCHUNK:
                wait_idx(k + 1, (k + 1) % 2)
                issue_main(k + 1, (k + 1) % 2, (k + 1) % 3)
        else:
            wait_scatter((i - 2) % 3)
            wait_idx(k + 1, (i + 1) % 2)
            issue_main(k + 1, (i + 1) % 2, (i + 1) % 3)

        wait_main(k, p2, p3)
        mult_scatter(p2, p3)

        if isinstance(k, int):
            if k + 2 < NCHUNK:
                issue_idx(k + 2, p2)
        else:
            issue_idx(k + 2, p2)

    issue_idx(0, 0)
    wait_idx(0, 0)
    issue_main(0, 0, 0)
    issue_idx(1, 1)
    for k in range(2):
        step(k, k)

    def six(b, carry):
        k = 6 * b + 2
        for i in range(6):
            step(k + i, (2 + i) % 6)
        return carry

    nblk6 = (NCHUNK - 2 - 4) // 6
    lax.fori_loop(0, nblk6, six, 0)
    for k in range(2 + nblk6 * 6, NCHUNK):
        step(k, k)

    wait_scatter((NCHUNK - 2) % 3)
    wait_scatter((NCHUNK - 1) % 3)
    plsc.subcore_barrier()

    pltpu.sync_copy(agg_sh.at[pl.ds(row0, ROWS_PT)],
                    out_hbm.at[cid, pl.ds(row0, ROWS_PT)])


def _conv_call(h, wi, a0c, a1c):
    kfn = pl.kernel(
        _conv_body,
        out_type=jax.ShapeDtypeStruct((SC_CORES, NROWS, NF), jnp.float32),
        mesh=_sc_mesh,
        compiler_params=_sc_params,
        scratch_types=[
            pltpu.VMEM_SHARED((NROWS, NF), jnp.float32),
            pltpu.VMEM((CHUNK,), jnp.int32),
            pltpu.VMEM((CHUNK,), jnp.int32),
            pltpu.VMEM((CHUNK,), jnp.int32),
            pltpu.VMEM((CHUNK,), jnp.int32),
            pltpu.VMEM((CHUNK, NF), jnp.float32),
            pltpu.VMEM((CHUNK, NF), jnp.float32),
            pltpu.VMEM((CHUNK, NF), jnp.float32),
            pltpu.VMEM((CHUNK, NF), jnp.bfloat16),
            pltpu.VMEM((CHUNK, NF), jnp.bfloat16),
            pltpu.VMEM((CHUNK, NF), jnp.bfloat16),
            pltpu.VMEM((CHUNK,), jnp.int32),
            pltpu.VMEM((CHUNK,), jnp.int32),
            pltpu.VMEM((CHUNK,), jnp.int32),
            pltpu.VMEM((ZROWS, NF), jnp.float32),
            pltpu.SemaphoreType.DMA,
            pltpu.SemaphoreType.DMA,
            pltpu.SemaphoreType.DMA,
            pltpu.SemaphoreType.DMA,
            pltpu.SemaphoreType.DMA,
            pltpu.SemaphoreType.DMA,
            pltpu.SemaphoreType.DMA,
            pltpu.SemaphoreType.DMA,
            pltpu.SemaphoreType.DMA,
            pltpu.SemaphoreType.DMA,
            pltpu.SemaphoreType.DMA,
            pltpu.SemaphoreType.DMA,
            pltpu.SemaphoreType.DMA,
        ],
    )
    return kfn(h, wi, a0c, a1c)


def _embed_body(z_ref, embed_ref, win0_ref, r_ref, h_ref):
    z = z_ref[:, :]
    classes = lax.broadcasted_iota(jnp.int32, (z.shape[0], 100), 1)
    onehot = (z == classes).astype(jnp.float32)
    r = jnp.dot(onehot, embed_ref[:, :], preferred_element_type=jnp.float32)
    r_ref[:, :] = r
    h_ref[:, :] = jnp.dot(r, win0_ref[:, :], preferred_element_type=jnp.float32)


def _embed_call(z, embed, win0):
    return pl.pallas_call(
        _embed_body,
        out_shape=(
            jax.ShapeDtypeStruct((N, NB), jnp.float32),
            jax.ShapeDtypeStruct((N, NF), jnp.float32),
        ),
    )(z.reshape(N, 1), embed, win0)


def _filter_body(e2_ref, wf1_ref, bf1_ref, wf2_ref, bf2_ref, out_ref):
    width = CUTOFF / (NG - 1)
    e2 = e2_ref[0, 0, :]
    e = jnp.sqrt(e2 + 1e-12)
    offs = lax.broadcasted_iota(jnp.int32, (e.shape[0], NG), 1).astype(jnp.float32) * width
    t = (e[:, None] - offs) * (1.0 / width)
    g = jnp.exp(-0.5 * t * t)
    pre = jnp.dot(g, wf1_ref[:, :], preferred_element_type=jnp.float32) + bf1_ref[0, :][None, :]
    w = jnp.dot(_ssp(pre), wf2_ref[:, :], preferred_element_type=jnp.float32) + bf2_ref[0, :][None, :]
    out_ref[:, :] = w.astype(jnp.bfloat16)


def _filter_call(e2r, wf1, bf1, wf2, bf2, be):
    nblk = EPAD // be
    return pl.pallas_call(
        _filter_body,
        grid=(nblk,),
        in_specs=[
            pl.BlockSpec((1, 1, be), lambda j: (j, 0, 0)),
            pl.BlockSpec((NG, NF), lambda j: (0, 0)),
            pl.BlockSpec((1, NF), lambda j: (0, 0)),
            pl.BlockSpec((NF, NF), lambda j: (0, 0)),
            pl.BlockSpec((1, NF), lambda j: (0, 0)),
        ],
        out_specs=pl.BlockSpec((be, NF), lambda j: (j, 0)),
        out_shape=jax.ShapeDtypeStruct((EPAD, NF), jnp.bfloat16),
    )(e2r, wf1, bf1.reshape(1, NF), wf2, bf2.reshape(1, NF))


def _update_body(p_ref, r_ref, wout_ref, bout_ref, win_ref, rn_ref, hn_ref):
    agg = p_ref[0, :N, :] + p_ref[1, :N, :]
    dr = jnp.dot(_ssp(agg), wout_ref[:, :], preferred_element_type=jnp.float32) + bout_ref[0, :][None, :]
    rn = r_ref[:, :] + dr
    rn_ref[:, :] = rn
    hn_ref[:, :] = jnp.dot(rn, win_ref[:, :], preferred_element_type=jnp.float32)


def _update_call(parts, r, wout, bout, win_next):
    return pl.pallas_call(
        _update_body,
        out_shape=(
            jax.ShapeDtypeStruct((N, NB), jnp.float32),
            jax.ShapeDtypeStruct((N, NF), jnp.float32),
        ),
    )(parts, r, wout, bout.reshape(1, NB), win_next)


def _readout_body(r_ref, wr1_ref, br1_ref, wr2_ref, br2_ref, out_ref):
    hidden = _ssp(jnp.dot(r_ref[:, :], wr1_ref[:, :], preferred_element_type=jnp.float32)
                  + br1_ref[0, :][None, :])
    atom_e = jnp.dot(hidden, wr2_ref[:, :], preferred_element_type=jnp.float32) + br2_ref[0, 0]
    per_mol = jnp.sum(atom_e.reshape(NMOL, N // NMOL), axis=1)
    out_ref[:] = per_mol


def _readout_call(r, Wr1, br1, Wr2, br2):
    return pl.pallas_call(
        _readout_body,
        out_shape=jax.ShapeDtypeStruct((NMOL,), jnp.float32),
    )(r, Wr1, br1.reshape(1, 64), Wr2, br2.reshape(1, 1))


def kernel(z, xyz, nbr_list, mol_ids, embed, Win, Wf1, bf1, Wf2, bf2, Wout, bout, Wr1, br1, Wr2, br2):
    a0 = nbr_list[:, 0].astype(jnp.int32)
    a1 = nbr_list[:, 1].astype(jnp.int32)
    npad = EPAD - E
    trash = N + (jnp.arange(npad, dtype=jnp.int32) % (NROWS - N))
    a0p = jnp.roll(jnp.concatenate([a0, trash]), npad // 2)
    a1p = jnp.roll(jnp.concatenate([a1, jnp.arange(npad, dtype=jnp.int32) % N]), npad // 2)
    a0c = a0p.reshape(NW, NCHUNK, CHUNK)
    a1c = a1p.reshape(NW, NCHUNK, CHUNK)
    a0d = a0p.reshape(NW, EPW // 16, 16)
    a1d = a1p.reshape(NW, EPW // 16, 16)

    xyzp = jnp.zeros((NROWS, 3), jnp.float32).at[:N].set(xyz)
    e2 = _dist_call(xyzp[:, 0], xyzp[:, 1], xyzp[:, 2], a0d, a1d)
    e2r = e2.reshape(EPAD // 2048, 1, 2048)

    grp = jnp.arange(NF // 32) * 32
    ev = grp[:, None] + 2 * jnp.arange(16)[None, :]
    perm = jnp.concatenate([ev, ev + 1], axis=1).reshape(NF)

    r, h = _embed_call(z, embed, Win[0][:, perm])

    for i in range(NC):
        wi = _filter_call(e2r, Wf1[i], bf1[i], Wf2[i], bf2[i], 2048)
        parts = _conv_call(h, wi, a0c, a1c)
        win_next = (Win[i + 1][:, perm] if i + 1 < NC
                    else jnp.zeros((NB, NF), jnp.float32))
        r, h = _update_call(parts, r, Wout[i][perm], bout[i], win_next)

    return _readout_call(r, Wr1, br1, Wr2, br2)


# speedup vs baseline: 1.8898x; 1.8898x over previous
import functools

import jax
import jax.numpy as jnp
from jax import lax
from jax.experimental import pallas as pl
from jax.experimental.pallas import tpu as pltpu
from jax.experimental.pallas import tpu_sc as plsc

N = 10000
E = 320000
NB = 128
NF = 128
NG = 32
NC = 3
NMOL = 100
CUTOFF = 5.0

LOG2 = 0.6931471805599453

SC_CORES = 2
SC_TILES = 16
NW = SC_CORES * SC_TILES
EPAD = 327680
EPW = EPAD // NW
CHUNK = 64
NCHUNK = EPW // CHUNK
NROWS = 10048
ROWS_PT = NROWS // SC_TILES
ZROWS = 8

_sc_mesh = plsc.VectorSubcoreMesh(core_axis_name="c", subcore_axis_name="s")
_sc_params = pltpu.CompilerParams(needs_layout_passes=False, use_tc_tiling_on_sc=False)


def _ssp(x):
    return jnp.maximum(x, 0.0) + jnp.log1p(jnp.exp(-jnp.abs(x))) - LOG2


def _dist_body(xs_hbm, ys_hbm, zs_hbm, a0_hbm, a1_hbm, out_hbm, xs_v, ys_v, zs_v, i0_v, i1_v, e2_v):
    cid = lax.axis_index("c")
    sid = lax.axis_index("s")
    wid = cid * SC_TILES + sid

    pltpu.sync_copy(xs_hbm, xs_v)
    pltpu.sync_copy(ys_hbm, ys_v)
    pltpu.sync_copy(zs_hbm, zs_v)
    pltpu.sync_copy(a0_hbm.at[wid], i0_v)
    pltpu.sync_copy(a1_hbm.at[wid], i1_v)

    def step(i, carry):
        a0 = i0_v[i, :]
        a1 = i1_v[i, :]
        dx = plsc.load_gather(xs_v, [a0]) - plsc.load_gather(xs_v, [a1])
        dy = plsc.load_gather(ys_v, [a0]) - plsc.load_gather(ys_v, [a1])
        dz = plsc.load_gather(zs_v, [a0]) - plsc.load_gather(zs_v, [a1])
        e2_v[i, :] = dx * dx + dy * dy + dz * dz
        return carry

    lax.fori_loop(0, EPW // 16, step, 0)
    pltpu.sync_copy(e2_v, out_hbm.at[wid])


def _dist_call(xs, ys, zs, a0r, a1r):
    kfn = pl.kernel(
        _dist_body,
        out_type=jax.ShapeDtypeStruct((NW, EPW // 16, 16), jnp.float32),
        mesh=_sc_mesh,
        compiler_params=_sc_params,
        scratch_types=[
            pltpu.VMEM((NROWS,), jnp.float32),
            pltpu.VMEM((NROWS,), jnp.float32),
            pltpu.VMEM((NROWS,), jnp.float32),
            pltpu.VMEM((EPW // 16, 16), jnp.int32),
            pltpu.VMEM((EPW // 16, 16), jnp.int32),
            pltpu.VMEM((EPW // 16, 16), jnp.float32),
        ],
    )
    return kfn(xs, ys, zs, a0r, a1r)


def _conv_body(h_hbm, wi_hbm, a0_hbm, a1_hbm, out_hbm, agg_sh,
               i0a, i0b, i1a, i1b, hg0, hg1, hg2, wv0, wv1, wv2,
               sx0, sx1, sx2, zb_v,
               sg0, sg1, sg2, sw0, sw1, sw2, ss0, ss1, ss2, q0a, q0b, q1a, q1b):
    cid = lax.axis_index("c")
    sid = lax.axis_index("s")
    wid = cid * SC_TILES + sid

    def zrow(r, carry):
        for c in range(NF // 16):
            zb_v[r, pl.ds(c * 16, 16)] = jnp.zeros((16,), jnp.float32)
        return carry

    lax.fori_loop(0, ZROWS, zrow, 0)

    row0 = sid * ROWS_PT

    def zblk(k, carry):
        pltpu.sync_copy(zb_v, agg_sh.at[pl.ds(row0 + k * ZROWS, ZROWS)])
        return carry

    lax.fori_loop(0, ROWS_PT // ZROWS, zblk, 0)

    if ROWS_PT % ZROWS != 0:
        tail = ROWS_PT % ZROWS
        pltpu.sync_copy(zb_v.at[pl.ds(0, tail)],
                        agg_sh.at[pl.ds(row0 + (ROWS_PT // ZROWS) * ZROWS, tail)])

    plsc.subcore_barrier()

    ebase = wid * EPW
    ibufs = ((i0a, i1a, q0a, q1a), (i0b, i1b, q0b, q1b))
    dbufs = ((hg0, wv0, sx0, sg0, sw0, ss0),
             (hg1, wv1, sx1, sg1, sw1, ss1),
             (hg2, wv2, sx2, sg2, sw2, ss2))

    def issue_idx(ch, p2):
        i0, i1, q0, q1 = ibufs[p2]
        pltpu.async_copy(a0_hbm.at[wid, ch], i0, q0)
        pltpu.async_copy(a1_hbm.at[wid, ch], i1, q1)

    def wait_idx(ch, p2):
        i0, i1, q0, q1 = ibufs[p2]
        pltpu.make_async_copy(a0_hbm.at[wid, ch], i0, q0).wait()
        pltpu.make_async_copy(a1_hbm.at[wid, ch], i1, q1).wait()

    def issue_main(ch, p2, p3):
        i1 = ibufs[p2][1]
        hg, wv, _, sg, sw, _ = dbufs[p3]
        pltpu.async_copy(h_hbm.at[i1], hg, sg)
        pltpu.async_copy(wi_hbm.at[pl.ds(ebase + ch * CHUNK, CHUNK)], wv, sw)

    def wait_main(ch, p2, p3):
        i1 = ibufs[p2][1]
        hg, wv, _, sg, sw, _ = dbufs[p3]
        pltpu.make_async_copy(h_hbm.at[i1], hg, sg).wait()
        pltpu.make_async_copy(wi_hbm.at[pl.ds(ebase + ch * CHUNK, CHUNK)], wv, sw).wait()

    def wait_scatter(p3):
        hg, wv, sx, _, _, ss = dbufs[p3]
        pltpu.make_async_copy(wv, agg_sh.at[sx], ss).wait()

    def mult_scatter(p2, p3):
        i0 = ibufs[p2][0]
        hg, wv, sx, _, _, ss = dbufs[p3]

        def mrow(r, c2):
            for c in range(NF // 16):
                s = pl.ds(c * 16, 16)
                wv[r, s] = wv[r, s] * hg[r, s]
            return c2

        lax.fori_loop(0, CHUNK, mrow, 0)
        for c in range(CHUNK // 16):
            s = pl.ds(c * 16, 16)
            sx[s] = i0[s]
        pltpu.async_copy(wv, agg_sh.at[sx], ss, add=True)

    def step(k, i):
        p3 = i % 3
        p2 = i % 2

        if isinstance(k, int):
            if k >= 2:
                wait_scatter((k - 2) % 3)
            if k + 1 < NCHUNK:
                wait_idx(k + 1, (k + 1) % 2)
                issue_main(k + 1, (k + 1) % 2, (k + 1) % 3)
        else:
            wait_scatter((i - 2) % 3)
            wait_idx(k + 1, (i + 1) % 2)
            issue_main(k + 1, (i + 1) % 2, (i + 1) % 3)

        wait_main(k, p2, p3)
        mult_scatter(p2, p3)

        if isinstance(k, int):
            if k + 2 < NCHUNK:
                issue_idx(k + 2, p2)
        else:
            issue_idx(k + 2, p2)

    issue_idx(0, 0)
    wait_idx(0, 0)
    issue_main(0, 0, 0)
    issue_idx(1, 1)
    for k in range(2):
        step(k, k)

    def six(b, carry):
        k = 6 * b + 2
        for i in range(6):
            step(k + i, (2 + i) % 6)
        return carry

    nblk6 = (NCHUNK - 2 - 4) // 6
    lax.fori_loop(0, nblk6, six, 0)
    for k in range(2 + nblk6 * 6, NCHUNK):
        step(k, k)

    wait_scatter((NCHUNK - 2) % 3)
    wait_scatter((NCHUNK - 1) % 3)
    plsc.subcore_barrier()

    pltpu.sync_copy(agg_sh.at[pl.ds(row0, ROWS_PT)],
                    out_hbm.at[cid, pl.ds(row0, ROWS_PT)])


def _conv_call(h, wi, a0c, a1c):
    kfn = pl.kernel(
        _conv_body,
        out_type=jax.ShapeDtypeStruct((SC_CORES, NROWS, NF), jnp.float32),
        mesh=_sc_mesh,
        compiler_params=_sc_params,
        scratch_types=[
            pltpu.VMEM_SHARED((NROWS, NF), jnp.float32),
            pltpu.VMEM((CHUNK,), jnp.int32),
            pltpu.VMEM((CHUNK,), jnp.int32),
            pltpu.VMEM((CHUNK,), jnp.int32),
            pltpu.VMEM((CHUNK,), jnp.int32),
            pltpu.VMEM((CHUNK, NF), jnp.float32),
            pltpu.VMEM((CHUNK, NF), jnp.float32),
            pltpu.VMEM((CHUNK, NF), jnp.float32),
            pltpu.VMEM((CHUNK, NF), jnp.float32),
            pltpu.VMEM((CHUNK, NF), jnp.float32),
            pltpu.VMEM((CHUNK, NF), jnp.float32),
            pltpu.VMEM((CHUNK,), jnp.int32),
            pltpu.VMEM((CHUNK,), jnp.int32),
            pltpu.VMEM((CHUNK,), jnp.int32),
            pltpu.VMEM((ZROWS, NF), jnp.float32),
            pltpu.SemaphoreType.DMA,
            pltpu.SemaphoreType.DMA,
            pltpu.SemaphoreType.DMA,
            pltpu.SemaphoreType.DMA,
            pltpu.SemaphoreType.DMA,
            pltpu.SemaphoreType.DMA,
            pltpu.SemaphoreType.DMA,
            pltpu.SemaphoreType.DMA,
            pltpu.SemaphoreType.DMA,
            pltpu.SemaphoreType.DMA,
            pltpu.SemaphoreType.DMA,
            pltpu.SemaphoreType.DMA,
            pltpu.SemaphoreType.DMA,
        ],
    )
    return kfn(h, wi, a0c, a1c)


def _embed_body(z_ref, embed_ref, win0_ref, r_ref, h_ref):
    z = z_ref[:, :]
    classes = lax.broadcasted_iota(jnp.int32, (z.shape[0], 100), 1)
    onehot = (z == classes).astype(jnp.float32)
    r = jnp.dot(onehot, embed_ref[:, :], preferred_element_type=jnp.float32)
    r_ref[:, :] = r
    h_ref[:, :] = jnp.dot(r, win0_ref[:, :], preferred_element_type=jnp.float32)


def _embed_call(z, embed, win0):
    return pl.pallas_call(
        _embed_body,
        out_shape=(
            jax.ShapeDtypeStruct((N, NB), jnp.float32),
            jax.ShapeDtypeStruct((N, NF), jnp.float32),
        ),
    )(z.reshape(N, 1), embed, win0)


def _filter_body(e2_ref, wf1_ref, bf1_ref, wf2_ref, bf2_ref, out_ref):
    width = CUTOFF / (NG - 1)
    e2 = e2_ref[0, 0, :]
    e = jnp.sqrt(e2 + 1e-12)
    offs = lax.broadcasted_iota(jnp.int32, (e.shape[0], NG), 1).astype(jnp.float32) * width
    t = (e[:, None] - offs) * (1.0 / width)
    g = jnp.exp(-0.5 * t * t)
    pre = jnp.dot(g, wf1_ref[:, :], preferred_element_type=jnp.float32) + bf1_ref[0, :][None, :]
    w = jnp.dot(_ssp(pre), wf2_ref[:, :], preferred_element_type=jnp.float32) + bf2_ref[0, :][None, :]
    out_ref[:, :] = w


def _filter_call(e2r, wf1, bf1, wf2, bf2, be):
    nblk = EPAD // be
    return pl.pallas_call(
        _filter_body,
        grid=(nblk,),
        in_specs=[
            pl.BlockSpec((1, 1, be), lambda j: (j, 0, 0)),
            pl.BlockSpec((NG, NF), lambda j: (0, 0)),
            pl.BlockSpec((1, NF), lambda j: (0, 0)),
            pl.BlockSpec((NF, NF), lambda j: (0, 0)),
            pl.BlockSpec((1, NF), lambda j: (0, 0)),
        ],
        out_specs=pl.BlockSpec((be, NF), lambda j: (j, 0)),
        out_shape=jax.ShapeDtypeStruct((EPAD, NF), jnp.float32),
    )(e2r, wf1, bf1.reshape(1, NF), wf2, bf2.reshape(1, NF))


def _update_body(p_ref, r_ref, wout_ref, bout_ref, win_ref, rn_ref, hn_ref):
    agg = p_ref[0, :N, :] + p_ref[1, :N, :]
    dr = jnp.dot(_ssp(agg), wout_ref[:, :], preferred_element_type=jnp.float32) + bout_ref[0, :][None, :]
    rn = r_ref[:, :] + dr
    rn_ref[:, :] = rn
    hn_ref[:, :] = jnp.dot(rn, win_ref[:, :], preferred_element_type=jnp.float32)


def _update_call(parts, r, wout, bout, win_next):
    return pl.pallas_call(
        _update_body,
        out_shape=(
            jax.ShapeDtypeStruct((N, NB), jnp.float32),
            jax.ShapeDtypeStruct((N, NF), jnp.float32),
        ),
    )(parts, r, wout, bout.reshape(1, NB), win_next)


def _readout_body(r_ref, wr1_ref, br1_ref, wr2_ref, br2_ref, out_ref):
    hidden = _ssp(jnp.dot(r_ref[:, :], wr1_ref[:, :], preferred_element_type=jnp.float32)
                  + br1_ref[0, :][None, :])
    atom_e = jnp.dot(hidden, wr2_ref[:, :], preferred_element_type=jnp.float32) + br2_ref[0, 0]
    per_mol = jnp.sum(atom_e.reshape(NMOL, N // NMOL), axis=1)
    out_ref[:] = per_mol


def _readout_call(r, Wr1, br1, Wr2, br2):
    return pl.pallas_call(
        _readout_body,
        out_shape=jax.ShapeDtypeStruct((NMOL,), jnp.float32),
    )(r, Wr1, br1.reshape(1, 64), Wr2, br2.reshape(1, 1))


def kernel(z, xyz, nbr_list, mol_ids, embed, Win, Wf1, bf1, Wf2, bf2, Wout, bout, Wr1, br1, Wr2, br2):
    a0 = nbr_list[:, 0].astype(jnp.int32)
    a1 = nbr_list[:, 1].astype(jnp.int32)
    npad = EPAD - E
    trash = N + (jnp.arange(npad, dtype=jnp.int32) % (NROWS - N))
    a0p = jnp.roll(jnp.concatenate([a0, trash]), npad // 2)
    a1p = jnp.roll(jnp.concatenate([a1, jnp.arange(npad, dtype=jnp.int32) % N]), npad // 2)
    a0c = a0p.reshape(NW, NCHUNK, CHUNK)
    a1c = a1p.reshape(NW, NCHUNK, CHUNK)
    a0d = a0p.reshape(NW, EPW // 16, 16)
    a1d = a1p.reshape(NW, EPW // 16, 16)

    xyzp = jnp.zeros((NROWS, 3), jnp.float32).at[:N].set(xyz)
    e2 = _dist_call(xyzp[:, 0], xyzp[:, 1], xyzp[:, 2], a0d, a1d)
    e2r = e2.reshape(EPAD // 2048, 1, 2048)

    r, h = _embed_call(z, embed, Win[0])

    for i in range(NC):
        wi = _filter_call(e2r, Wf1[i], bf1[i], Wf2[i], bf2[i], 2048)
        parts = _conv_call(h, wi, a0c, a1c)
        win_next = Win[i + 1] if i + 1 < NC else jnp.zeros((NB, NF), jnp.float32)
        r, h = _update_call(parts, r, Wout[i], bout[i], win_next)

    return _readout_call(r, Wr1, br1, Wr2, br2)
